# baseline (speedup 1.0000x reference)
import functools

import jax
import jax.numpy as jnp
from jax import lax
from jax.experimental import pallas as pl
from jax.experimental.pallas import tpu as pltpu
from jax.experimental.pallas import tpu_sc as plsc

NUM_CLASSES = 1000
N_CTX = 16
CTX_DIM = 512
SEQ_LEN = 77
SUFFIX_LEN = SEQ_LEN - 1 - N_CTX
BATCH = 4096

CTX_ROW = N_CTX * CTX_DIM
SUF_ROW = SUFFIX_LEN * CTX_DIM
OUT_ROW = SEQ_LEN * CTX_DIM

_INFO = plsc.get_sparse_core_info()
_NC = _INFO.num_cores
_NW = _INFO.num_cores * _INFO.num_subcores
_B_PER_W = BATCH // _NW

_MESH = plsc.VectorSubcoreMesh(core_axis_name="c", subcore_axis_name="s")


@functools.partial(
    pl.kernel,
    out_type=jax.ShapeDtypeStruct((BATCH * OUT_ROW,), jnp.float32),
    mesh=_MESH,
    scratch_types=[
        pltpu.VMEM((_B_PER_W,), jnp.int32),
        pltpu.SMEM((_B_PER_W,), jnp.int32),
    ],
)
def _assemble(label_hbm, ctx_hbm, pref_hbm, suf_hbm, out_hbm, idx_v, idx_s):
    wid = lax.axis_index("s") * _NC + lax.axis_index("c")
    base = wid * _B_PER_W
    pltpu.sync_copy(label_hbm.at[pl.ds(base, _B_PER_W)], idx_v)

    def group(g, carry):
        lbls = idx_v[pl.ds(g * 16, 16)]
        for j in range(16):
            lbl = lbls[j]
            out_off = (base + g * 16 + j) * OUT_ROW
            pltpu.sync_copy(pref_hbm, out_hbm.at[pl.ds(out_off, CTX_DIM)])
            pltpu.sync_copy(
                ctx_hbm.at[pl.ds(lbl * CTX_ROW, CTX_ROW)],
                out_hbm.at[pl.ds(out_off + CTX_DIM, CTX_ROW)],
            )
            pltpu.sync_copy(
                suf_hbm.at[pl.ds(lbl * SUF_ROW, SUF_ROW)],
                out_hbm.at[pl.ds(out_off + CTX_DIM + CTX_ROW, SUF_ROW)],
            )
        return carry

    lax.fori_loop(0, _B_PER_W // 16, group, 0)


def kernel(label, cls_ctx, token_prefix, token_suffix):
    out_flat = _assemble(
        label.astype(jnp.int32),
        cls_ctx.reshape(-1),
        token_prefix.reshape(-1),
        token_suffix.reshape(-1),
    )
    return out_flat.reshape(BATCH, SEQ_LEN, CTX_DIM)


# speedup vs baseline: 1.0000x; 1.0000x over previous
import functools

import jax
import jax.numpy as jnp
from jax import lax
from jax.experimental import pallas as pl
from jax.experimental.pallas import tpu as pltpu
from jax.experimental.pallas import tpu_sc as plsc

NUM_CLASSES = 1000
N_CTX = 16
CTX_DIM = 512
SEQ_LEN = 77
SUFFIX_LEN = SEQ_LEN - 1 - N_CTX
BATCH = 4096

CTX_ROW = N_CTX * CTX_DIM
SUF_ROW = SUFFIX_LEN * CTX_DIM
OUT_ROW = SEQ_LEN * CTX_DIM

_INFO = plsc.get_sparse_core_info()
_NC = _INFO.num_cores
_NW = _INFO.num_cores * _INFO.num_subcores
_B_PER_W = BATCH // _NW

_MESH = plsc.VectorSubcoreMesh(core_axis_name="c", subcore_axis_name="s")


@functools.partial(
    pl.kernel,
    out_type=jax.ShapeDtypeStruct((BATCH * OUT_ROW,), jnp.float32),
    mesh=_MESH,
    scratch_types=[
        pltpu.VMEM((_B_PER_W,), jnp.int32),
        pltpu.SemaphoreType.DMA,
    ],
)
def _assemble(label_hbm, ctx_hbm, pref_hbm, suf_hbm, out_hbm, idx_v, sem):
    wid = lax.axis_index("s") * _NC + lax.axis_index("c")
    base = wid * _B_PER_W
    pltpu.sync_copy(label_hbm.at[pl.ds(base, _B_PER_W)], idx_v)

    DEPTH = 10
    pending = []
    lbls = None
    for i in range(_B_PER_W):
        if i % 16 == 0:
            lbls = idx_v[pl.ds(i, 16)]
        lbl = lbls[i % 16]
        out_off = (base + i) * OUT_ROW
        pending.append(pltpu.async_copy(
            pref_hbm, out_hbm.at[pl.ds(out_off, CTX_DIM)], sem))
        pending.append(pltpu.async_copy(
            ctx_hbm.at[pl.ds(lbl * CTX_ROW, CTX_ROW)],
            out_hbm.at[pl.ds(out_off + CTX_DIM, CTX_ROW)], sem))
        pending.append(pltpu.async_copy(
            suf_hbm.at[pl.ds(lbl * SUF_ROW, SUF_ROW)],
            out_hbm.at[pl.ds(out_off + CTX_DIM + CTX_ROW, SUF_ROW)], sem))
        while len(pending) > 3 * DEPTH:
            pending.pop(0).wait()
    for copy in pending:
        copy.wait()


def kernel(label, cls_ctx, token_prefix, token_suffix):
    out_flat = _assemble(
        label.astype(jnp.int32),
        cls_ctx.reshape(-1),
        token_prefix.reshape(-1),
        token_suffix.reshape(-1),
    )
    return out_flat.reshape(BATCH, SEQ_LEN, CTX_DIM)


# speedup vs baseline: 10.6013x; 10.6010x over previous
import functools

import jax
import jax.numpy as jnp
from jax import lax
from jax.experimental import pallas as pl
from jax.experimental.pallas import tpu as pltpu
from jax.experimental.pallas import tpu_sc as plsc

NUM_CLASSES = 1000
N_CTX = 16
CTX_DIM = 512
SEQ_LEN = 77
SUFFIX_LEN = SEQ_LEN - 1 - N_CTX
BATCH = 4096

CTX_ROW = N_CTX * CTX_DIM
SUF_ROW = SUFFIX_LEN * CTX_DIM
OUT_ROW = SEQ_LEN * CTX_DIM

_INFO = plsc.get_sparse_core_info()
_NC = _INFO.num_cores
_NW = _INFO.num_cores * _INFO.num_subcores
_B_PER_W = BATCH // _NW
_NSLOT = 3


@functools.partial(
    pl.kernel,
    out_type=jax.ShapeDtypeStruct((BATCH * OUT_ROW,), jnp.float32),
    mesh=plsc.VectorSubcoreMesh(core_axis_name="c", subcore_axis_name="s"),
    scratch_types=[
        pltpu.VMEM((_B_PER_W,), jnp.int32),
        [pltpu.VMEM((OUT_ROW,), jnp.float32) for _ in range(_NSLOT)],
        [pltpu.SemaphoreType.DMA for _ in range(_NSLOT)],
        [pltpu.SemaphoreType.DMA for _ in range(_NSLOT)],
    ],
)
def _assemble(label_hbm, ctx_hbm, pref_hbm, suf_hbm, out_hbm,
              idx_v, rowbufs, gsems, ssems):
    wid = lax.axis_index("s") * _NC + lax.axis_index("c")
    base = wid * _B_PER_W
    pltpu.sync_copy(label_hbm.at[pl.ds(base, _B_PER_W)], idx_v)

    pref_copies = [
        pltpu.async_copy(pref_hbm, rowbufs[s].at[pl.ds(0, CTX_DIM)], gsems[s])
        for s in range(_NSLOT)
    ]
    for c in pref_copies:
        c.wait()

    store_pending = [None] * _NSLOT
    lbls = None
    for i in range(_B_PER_W):
        s = i % _NSLOT
        if store_pending[s] is not None:
            store_pending[s].wait()
        if i % 16 == 0:
            lbls = idx_v[pl.ds(i, 16)]
        lbl = lbls[i % 16]
        g1 = pltpu.async_copy(
            ctx_hbm.at[pl.ds(lbl * CTX_ROW, CTX_ROW)],
            rowbufs[s].at[pl.ds(CTX_DIM, CTX_ROW)], gsems[s])
        g2 = pltpu.async_copy(
            suf_hbm.at[pl.ds(lbl * SUF_ROW, SUF_ROW)],
            rowbufs[s].at[pl.ds(CTX_DIM + CTX_ROW, SUF_ROW)], gsems[s])
        g1.wait()
        g2.wait()
        store_pending[s] = pltpu.async_copy(
            rowbufs[s], out_hbm.at[pl.ds((base + i) * OUT_ROW, OUT_ROW)],
            ssems[s])
    for c in store_pending:
        if c is not None:
            c.wait()


def kernel(label, cls_ctx, token_prefix, token_suffix):
    out_flat = _assemble(
        label.astype(jnp.int32),
        cls_ctx.reshape(-1),
        token_prefix.reshape(-1),
        token_suffix.reshape(-1),
    )
    return out_flat.reshape(BATCH, SEQ_LEN, CTX_DIM)
